# Initial kernel scaffold; baseline (speedup 1.0000x reference)
import functools
import math

import jax
import jax.numpy as jnp
from jax import lax
from jax.experimental import pallas as pl
from jax.experimental.pallas import tpu as pltpu
from jax.experimental.pallas import tpu_sc as plsc


def _scale_table(embed_mat):
    V, D = embed_mat.shape
    scale = math.sqrt(D)
    blk = 8000
    if V % blk != 0:
        blk = V

    def body(t_ref, o_ref):
        v = t_ref[...]
        ss = jnp.sum(v * v, axis=1, keepdims=True)
        inv = scale * lax.rsqrt(jnp.maximum(ss, 1e-24))
        o_ref[...] = v * inv

    return pl.pallas_call(
        body,
        grid=(V // blk,),
        in_specs=[pl.BlockSpec((blk, D), lambda i: (i, 0))],
        out_specs=pl.BlockSpec((blk, D), lambda i: (i, 0)),
        out_shape=jax.ShapeDtypeStruct((V, D), jnp.float32),
    )(embed_mat)


def _sc_gather(table, idx_flat):
    V, D = table.shape
    (N,) = idx_flat.shape

    info = plsc.get_sparse_core_info()
    NC, NS = info.num_cores, info.num_subcores
    NW = NC * NS

    GW = 128
    K = 16
    C = K * GW
    assert N % (NW * C) == 0, (N, NW, C)
    per_w = N // NW
    n_chunks = per_w // C

    idx2d = idx_flat.reshape(N // GW, GW)
    mesh = plsc.VectorSubcoreMesh(core_axis_name="c", subcore_axis_name="s")

    @functools.partial(
        pl.kernel,
        out_type=jax.ShapeDtypeStruct((N, D), jnp.float32),
        mesh=mesh,
        scratch_types=[
            pltpu.VMEM((K, GW), jnp.int32),
            pltpu.VMEM((C, D), jnp.float32),
            pltpu.SemaphoreType.DMA,
        ],
    )
    def gather_kernel(table_hbm, idx_hbm, out_hbm, idx_v, rows_v, sem):
        wid = lax.axis_index("s") * NC + lax.axis_index("c")
        base_blk = wid * (per_w // GW)

        def body(i, carry):
            r0 = base_blk + i * K
            pltpu.sync_copy(idx_hbm.at[pl.ds(r0, K)], idx_v)
            cps = [
                pltpu.async_copy(
                    table_hbm.at[idx_v.at[j]],
                    rows_v.at[pl.ds(j * GW, GW)],
                    sem,
                )
                for j in range(K)
            ]
            for cp in cps:
                cp.wait()
            pltpu.sync_copy(rows_v, out_hbm.at[pl.ds(r0 * GW, C)])
            return carry

        lax.fori_loop(0, n_chunks, body, 0)

    return gather_kernel(table, idx2d)


def kernel(embed_mat, x):
    B, H = x.shape
    _, D = embed_mat.shape
    table = _scale_table(embed_mat)
    out = _sc_gather(table, x.reshape(-1).astype(jnp.int32))
    return out.reshape(B, H, D)


# speedup vs baseline: 4.2326x; 4.2326x over previous
import functools
import math

import jax
import jax.numpy as jnp
from jax import lax
from jax.experimental import pallas as pl
from jax.experimental.pallas import tpu as pltpu
from jax.experimental.pallas import tpu_sc as plsc


def _scale_table(embed_mat):
    V, D = embed_mat.shape
    scale = math.sqrt(D)
    blk = 8000
    if V % blk != 0:
        blk = V

    def body(t_ref, o_ref):
        v = t_ref[...]
        ss = jnp.sum(v * v, axis=1, keepdims=True)
        inv = scale * lax.rsqrt(jnp.maximum(ss, 1e-24))
        o_ref[...] = v * inv

    return pl.pallas_call(
        body,
        grid=(V // blk,),
        in_specs=[pl.BlockSpec((blk, D), lambda i: (i, 0))],
        out_specs=pl.BlockSpec((blk, D), lambda i: (i, 0)),
        out_shape=jax.ShapeDtypeStruct((V, D), jnp.float32),
    )(embed_mat)


def _sc_gather(table, idx_flat):
    V, D = table.shape
    (N,) = idx_flat.shape

    info = plsc.get_sparse_core_info()
    NC, NS = info.num_cores, info.num_subcores
    NW = NC * NS

    GW = 128
    K = 16
    C = K * GW
    assert N % (NW * C) == 0, (N, NW, C)
    per_w = N // NW
    n_chunks = per_w // C

    idx2d = idx_flat.reshape(N // GW, GW)
    mesh = plsc.VectorSubcoreMesh(core_axis_name="c", subcore_axis_name="s")

    @functools.partial(
        pl.kernel,
        out_type=jax.ShapeDtypeStruct((N, D), jnp.float32),
        mesh=mesh,
        scratch_types=[
            pltpu.VMEM((K, GW), jnp.int32),
            pltpu.VMEM((C, D), jnp.float32),
            pltpu.SemaphoreType.DMA,
        ],
        compiler_params=pltpu.CompilerParams(use_tc_tiling_on_sc=False),
    )
    def gather_kernel(table_hbm, idx_hbm, out_hbm, idx_v, rows_v, sem):
        wid = lax.axis_index("s") * NC + lax.axis_index("c")
        base_blk = wid * (per_w // GW)

        def body(i, carry):
            r0 = base_blk + i * K
            pltpu.sync_copy(idx_hbm.at[pl.ds(r0, K)], idx_v)
            cps = [
                pltpu.async_copy(
                    table_hbm.at[idx_v.at[j]],
                    rows_v.at[pl.ds(j * GW, GW)],
                    sem,
                )
                for j in range(K)
            ]
            for cp in cps:
                cp.wait()
            pltpu.sync_copy(rows_v, out_hbm.at[pl.ds(r0 * GW, C)])
            return carry

        lax.fori_loop(0, n_chunks, body, 0)

    return gather_kernel(table, idx2d)


def kernel(embed_mat, x):
    B, H = x.shape
    _, D = embed_mat.shape
    table = _scale_table(embed_mat)
    out = _sc_gather(table, x.reshape(-1).astype(jnp.int32))
    return out.reshape(B, H, D)


# speedup vs baseline: 4.2405x; 1.0019x over previous
import functools
import math

import jax
import jax.numpy as jnp
from jax import lax
from jax.experimental import pallas as pl
from jax.experimental.pallas import tpu as pltpu
from jax.experimental.pallas import tpu_sc as plsc


def _scale_table(embed_mat):
    V, D = embed_mat.shape
    scale = math.sqrt(D)
    blk = 8000
    if V % blk != 0:
        blk = V

    def body(t_ref, o_ref):
        v = t_ref[...]
        ss = jnp.sum(v * v, axis=1, keepdims=True)
        inv = scale * lax.rsqrt(jnp.maximum(ss, 1e-24))
        o_ref[...] = v * inv

    return pl.pallas_call(
        body,
        grid=(V // blk,),
        in_specs=[pl.BlockSpec((blk, D), lambda i: (i, 0))],
        out_specs=pl.BlockSpec((blk, D), lambda i: (i, 0)),
        out_shape=jax.ShapeDtypeStruct((V, D), jnp.float32),
    )(embed_mat)


def _sc_gather(table, idx_flat):
    V, D = table.shape
    (N,) = idx_flat.shape

    info = plsc.get_sparse_core_info()
    NC, NS = info.num_cores, info.num_subcores
    NW = NC * NS

    GW = 128
    K = 8
    C = K * GW
    NBUF = 2
    assert N % (NW * C * NBUF) == 0, (N, NW, C)
    per_w = N // NW
    n_outer = per_w // (C * NBUF)

    idx2d = idx_flat.reshape(N // GW, GW)
    mesh = plsc.VectorSubcoreMesh(core_axis_name="c", subcore_axis_name="s")

    @functools.partial(
        pl.kernel,
        out_type=jax.ShapeDtypeStruct((N, D), jnp.float32),
        mesh=mesh,
        scratch_types=[
            pltpu.VMEM((K, GW), jnp.int32),
            pltpu.VMEM((NBUF * C, D), jnp.float32),
            pltpu.SemaphoreType.DMA,
            pltpu.SemaphoreType.DMA,
            pltpu.SemaphoreType.DMA,
        ],
        compiler_params=pltpu.CompilerParams(use_tc_tiling_on_sc=False),
    )
    def gather_kernel(table_hbm, idx_hbm, out_hbm, idx_v, rows_v, semg,
                      semw0, semw1):
        wid = lax.axis_index("s") * NC + lax.axis_index("c")
        base_blk = wid * (per_w // GW)
        semw = (semw0, semw1)

        def outer(i, carry):
            for b in range(NBUF):
                rows_b = rows_v.at[pl.ds(b * C, C)]

                @pl.when(i >= 1)
                def _drain_prev():
                    pltpu.make_async_copy(
                        rows_b, out_hbm.at[pl.ds(0, C)], semw[b]).wait()

                r0 = base_blk + (i * NBUF + b) * K
                pltpu.sync_copy(idx_hbm.at[pl.ds(r0, K)], idx_v)
                cps = [
                    pltpu.async_copy(
                        table_hbm.at[idx_v.at[j]],
                        rows_v.at[pl.ds(b * C + j * GW, GW)],
                        semg,
                    )
                    for j in range(K)
                ]
                for cp in cps:
                    cp.wait()
                pltpu.async_copy(rows_b, out_hbm.at[pl.ds(r0 * GW, C)],
                                 semw[b])
            return carry

        lax.fori_loop(0, n_outer, outer, 0)
        for b in range(NBUF):
            pltpu.make_async_copy(
                rows_v.at[pl.ds(b * C, C)], out_hbm.at[pl.ds(0, C)],
                semw[b]).wait()

    return gather_kernel(table, idx2d)


def kernel(embed_mat, x):
    B, H = x.shape
    _, D = embed_mat.shape
    table = _scale_table(embed_mat)
    out = _sc_gather(table, x.reshape(-1).astype(jnp.int32))
    return out.reshape(B, H, D)


# speedup vs baseline: 4.8560x; 1.1451x over previous
import functools
import math

import jax
import jax.numpy as jnp
from jax import lax
from jax.experimental import pallas as pl
from jax.experimental.pallas import tpu as pltpu
from jax.experimental.pallas import tpu_sc as plsc


def _scale_table(embed_mat):
    V, D = embed_mat.shape
    scale = math.sqrt(D)
    tabT = embed_mat.T
    blk = min(V, 8192)

    def body(t_ref, o_ref):
        v = t_ref[...]
        ss = jnp.sum(v * v, axis=0, keepdims=True)
        inv = scale * lax.rsqrt(jnp.maximum(ss, 1e-24))
        o_ref[...] = jnp.transpose(v * inv, (1, 0))

    return pl.pallas_call(
        body,
        grid=(pl.cdiv(V, blk),),
        in_specs=[pl.BlockSpec((D, blk), lambda i: (0, i))],
        out_specs=pl.BlockSpec((blk, D), lambda i: (i, 0)),
        out_shape=jax.ShapeDtypeStruct((V, D), jnp.float32),
    )(tabT)


def _sc_gather(table, idx_flat):
    V, D = table.shape
    (N,) = idx_flat.shape

    info = plsc.get_sparse_core_info()
    NC, NS = info.num_cores, info.num_subcores
    NW = NC * NS

    GW = 128
    K = 8
    C = K * GW
    NBUF = 2
    assert N % (NW * C * NBUF) == 0, (N, NW, C)
    per_w = N // NW
    n_outer = per_w // (C * NBUF)

    idx2d = idx_flat.reshape(N // GW, GW)
    mesh = plsc.VectorSubcoreMesh(core_axis_name="c", subcore_axis_name="s")

    @functools.partial(
        pl.kernel,
        out_type=jax.ShapeDtypeStruct((N, D), jnp.float32),
        mesh=mesh,
        scratch_types=[
            pltpu.VMEM((K, GW), jnp.int32),
            pltpu.VMEM((NBUF * C, D), jnp.float32),
            pltpu.SemaphoreType.DMA,
            pltpu.SemaphoreType.DMA,
            pltpu.SemaphoreType.DMA,
        ],
        compiler_params=pltpu.CompilerParams(use_tc_tiling_on_sc=False),
    )
    def gather_kernel(table_hbm, idx_hbm, out_hbm, idx_v, rows_v, semg,
                      semw0, semw1):
        wid = lax.axis_index("s") * NC + lax.axis_index("c")
        base_blk = wid * (per_w // GW)
        semw = (semw0, semw1)

        def outer(i, carry):
            for b in range(NBUF):
                rows_b = rows_v.at[pl.ds(b * C, C)]

                @pl.when(i >= 1)
                def _drain_prev():
                    pltpu.make_async_copy(
                        rows_b, out_hbm.at[pl.ds(0, C)], semw[b]).wait()

                r0 = base_blk + (i * NBUF + b) * K
                pltpu.sync_copy(idx_hbm.at[pl.ds(r0, K)], idx_v)
                cps = [
                    pltpu.async_copy(
                        table_hbm.at[idx_v.at[j]],
                        rows_v.at[pl.ds(b * C + j * GW, GW)],
                        semg,
                    )
                    for j in range(K)
                ]
                for cp in cps:
                    cp.wait()
                pltpu.async_copy(rows_b, out_hbm.at[pl.ds(r0 * GW, C)],
                                 semw[b])
            return carry

        lax.fori_loop(0, n_outer, outer, 0)
        for b in range(NBUF):
            pltpu.make_async_copy(
                rows_v.at[pl.ds(b * C, C)], out_hbm.at[pl.ds(0, C)],
                semw[b]).wait()

    return gather_kernel(table, idx2d)


def kernel(embed_mat, x):
    B, H = x.shape
    _, D = embed_mat.shape
    table = _scale_table(embed_mat)
    out = _sc_gather(table, x.reshape(-1).astype(jnp.int32))
    return out.reshape(B, H, D)


# speedup vs baseline: 5.5539x; 1.1437x over previous
import functools
import math

import jax
import jax.numpy as jnp
from jax import lax
from jax.experimental import pallas as pl
from jax.experimental.pallas import tpu as pltpu
from jax.experimental.pallas import tpu_sc as plsc

_TBLK = 8192
_TQRT = 2048


def _scale_table(embed_mat):
    V, D = embed_mat.shape
    scale = math.sqrt(D)
    tabT = embed_mat.T
    blk = _TBLK
    qrt = _TQRT
    grid = pl.cdiv(V, blk)
    vpad = grid * blk
    nq = 128 // D

    def body(t_ref, o_ref):
        v = t_ref[...]
        ss = jnp.sum(v * v, axis=0, keepdims=True)
        inv = scale * lax.rsqrt(jnp.maximum(ss, 1e-24))
        z = v * inv
        parts = [
            jnp.transpose(z[:, c * qrt:(c + 1) * qrt], (1, 0))
            for c in range(nq)
        ]
        o_ref[...] = jnp.concatenate(parts, axis=1)

    out = pl.pallas_call(
        body,
        grid=(grid,),
        in_specs=[pl.BlockSpec((D, blk), lambda i: (0, i))],
        out_specs=pl.BlockSpec((qrt, 128), lambda i: (i, 0)),
        out_shape=jax.ShapeDtypeStruct((grid * qrt, 128), jnp.float32),
    )(tabT)
    return out.reshape(vpad, D)


def _sc_gather(table, idx_flat):
    V, D = table.shape
    (N,) = idx_flat.shape

    info = plsc.get_sparse_core_info()
    NC, NS = info.num_cores, info.num_subcores
    NW = NC * NS

    GW = 128
    K = 8
    C = K * GW
    NBUF = 2
    assert N % (NW * C * NBUF) == 0, (N, NW, C)
    per_w = N // NW
    n_outer = per_w // (C * NBUF)

    idx2d = idx_flat.reshape(N // GW, GW)
    mesh = plsc.VectorSubcoreMesh(core_axis_name="c", subcore_axis_name="s")

    @functools.partial(
        pl.kernel,
        out_type=jax.ShapeDtypeStruct((N, D), jnp.float32),
        mesh=mesh,
        scratch_types=[
            pltpu.VMEM((K, GW), jnp.int32),
            pltpu.VMEM((NBUF * C, D), jnp.float32),
            pltpu.SemaphoreType.DMA,
            pltpu.SemaphoreType.DMA,
            pltpu.SemaphoreType.DMA,
        ],
        compiler_params=pltpu.CompilerParams(use_tc_tiling_on_sc=False),
    )
    def gather_kernel(table_hbm, idx_hbm, out_hbm, idx_v, rows_v, semg,
                      semw0, semw1):
        wid = lax.axis_index("s") * NC + lax.axis_index("c")
        base_blk = wid * (per_w // GW)
        semw = (semw0, semw1)

        def outer(i, carry):
            for b in range(NBUF):
                rows_b = rows_v.at[pl.ds(b * C, C)]

                @pl.when(i >= 1)
                def _drain_prev():
                    pltpu.make_async_copy(
                        rows_b, out_hbm.at[pl.ds(0, C)], semw[b]).wait()

                r0 = base_blk + (i * NBUF + b) * K
                pltpu.sync_copy(idx_hbm.at[pl.ds(r0, K)], idx_v)
                for j in range(K):
                    for l in range(GW // 16):
                        w = idx_v[j, pl.ds(l * 16, 16)]
                        s = ((w & (-_TBLK)) | ((w & (_TQRT - 1)) << 2)
                             | ((w & (_TBLK - 1)) >> 11))
                        idx_v[j, pl.ds(l * 16, 16)] = s
                cps = [
                    pltpu.async_copy(
                        table_hbm.at[idx_v.at[j]],
                        rows_v.at[pl.ds(b * C + j * GW, GW)],
                        semg,
                    )
                    for j in range(K)
                ]
                for cp in cps:
                    cp.wait()
                pltpu.async_copy(rows_b, out_hbm.at[pl.ds(r0 * GW, C)],
                                 semw[b])
            return carry

        lax.fori_loop(0, n_outer, outer, 0)
        for b in range(NBUF):
            pltpu.make_async_copy(
                rows_v.at[pl.ds(b * C, C)], out_hbm.at[pl.ds(0, C)],
                semw[b]).wait()

    return gather_kernel(table, idx2d)


def kernel(embed_mat, x):
    B, H = x.shape
    _, D = embed_mat.shape
    table = _scale_table(embed_mat)
    out = _sc_gather(table, x.reshape(-1).astype(jnp.int32))
    return out.reshape(B, H, D)
